# baseline (speedup 1.0000x reference)
import jax
import jax.numpy as jnp
from jax.experimental import pallas as pl

NUM_PATCHES = 576
LATENT_DIM = 768
BATCH = 64


def _add_kernel(x_ref, pos_ref, out_ref):
    out_ref[...] = x_ref[...] + pos_ref[...]


def kernel(x, pos_table):
    return pl.pallas_call(
        _add_kernel,
        grid=(BATCH,),
        in_specs=[
            pl.BlockSpec((1, NUM_PATCHES, LATENT_DIM), lambda b: (b, 0, 0)),
            pl.BlockSpec((NUM_PATCHES, LATENT_DIM), lambda b: (0, 0)),
        ],
        out_specs=pl.BlockSpec((1, NUM_PATCHES, LATENT_DIM), lambda b: (b, 0, 0)),
        out_shape=jax.ShapeDtypeStruct((BATCH, NUM_PATCHES, LATENT_DIM), x.dtype),
    )(x, pos_table)


# speedup vs baseline: 1.1915x; 1.1915x over previous
import jax
import jax.numpy as jnp
from jax.experimental import pallas as pl

NUM_PATCHES = 576
LATENT_DIM = 768
BATCH = 64


BB = 4


def _add_kernel(x_ref, pos_ref, out_ref):
    out_ref[...] = x_ref[...] + pos_ref[...]


def kernel(x, pos_table):
    return pl.pallas_call(
        _add_kernel,
        grid=(BATCH // BB,),
        in_specs=[
            pl.BlockSpec((BB, NUM_PATCHES, LATENT_DIM), lambda b: (b, 0, 0)),
            pl.BlockSpec((NUM_PATCHES, LATENT_DIM), lambda b: (0, 0)),
        ],
        out_specs=pl.BlockSpec((BB, NUM_PATCHES, LATENT_DIM), lambda b: (b, 0, 0)),
        out_shape=jax.ShapeDtypeStruct((BATCH, NUM_PATCHES, LATENT_DIM), x.dtype),
    )(x, pos_table)


# speedup vs baseline: 1.2094x; 1.0150x over previous
import jax
import jax.numpy as jnp
from jax.experimental import pallas as pl

NUM_PATCHES = 576
LATENT_DIM = 768
BATCH = 64


BB = 8


def _add_kernel(x_ref, pos_ref, out_ref):
    out_ref[...] = x_ref[...] + pos_ref[...]


def kernel(x, pos_table):
    return pl.pallas_call(
        _add_kernel,
        grid=(BATCH // BB,),
        in_specs=[
            pl.BlockSpec((BB, NUM_PATCHES, LATENT_DIM), lambda b: (b, 0, 0)),
            pl.BlockSpec((NUM_PATCHES, LATENT_DIM), lambda b: (0, 0)),
        ],
        out_specs=pl.BlockSpec((BB, NUM_PATCHES, LATENT_DIM), lambda b: (b, 0, 0)),
        out_shape=jax.ShapeDtypeStruct((BATCH, NUM_PATCHES, LATENT_DIM), x.dtype),
    )(x, pos_table)
